# baseline (speedup 1.0000x reference)
import jax
import jax.numpy as jnp
import numpy as np
from jax.experimental import pallas as pl
from jax.experimental.pallas import tpu as pltpu

H, W = 37, 75
NT = H * W
NP = 2816


def _build_masks() -> np.ndarray:
    t = np.arange(NP)
    h, w = t // W, t % W
    rows = []
    for kh in range(3):
        for kw in range(3):
            di, dj = kh - 1, kw - 1
            valid = (h + di >= 0) & (h + di < H) & (w + dj >= 0) & (w + dj < W) & (t < NT)
            rows.append(valid.astype(np.float32))
    rows.extend(np.zeros(NP, np.float32) for _ in range(7))
    return np.stack(rows)


_MASKS = _build_masks()


def _conv3x3(x, w_ref, masks_ref, relu):
    acc = None
    for tap in range(9):
        di, dj = tap // 3 - 1, tap % 3 - 1
        delta = di * W + dj
        if delta == 0:
            xs = x
        else:
            xs = jnp.concatenate([x[:, delta:], x[:, :delta]], axis=1)
        m = masks_ref[pl.ds(tap, 1), :]
        y = jnp.dot(w_ref[0, tap], xs * m, preferred_element_type=jnp.float32)
        acc = y if acc is None else acc + y
    return jnp.maximum(acc, 0.0) if relu else acc


def _memory_body(qf_ref, keys_ref, keys_t_ref, masks_ref, w1_ref, w2_ref,
                 w3_ref, w4_ref, out_ref, tq_scratch):
    b = pl.program_id(0)
    qf = qf_ref[...]
    norm = jnp.sqrt(jnp.sum(qf * qf, axis=0, keepdims=True))
    qn = qf / jnp.maximum(norm, 1e-12)

    keys = keys_ref[...]
    k_norm = jnp.sqrt(jnp.sum(keys * keys, axis=1, keepdims=True))
    q_norm = jnp.sqrt(jnp.sum(qn * qn, axis=0, keepdims=True))
    dots = jnp.dot(keys, qn, preferred_element_type=jnp.float32)
    cos = dots / jnp.maximum(k_norm * q_norm, 1e-6)
    e = jnp.exp(cos - jnp.max(cos, axis=0, keepdims=True))
    score = e / jnp.sum(e, axis=0, keepdims=True)
    upd = jnp.dot(keys_t_ref[...], score, preferred_element_type=jnp.float32)

    x = jnp.where(b == 1, upd, qn)
    x = _conv3x3(x, w1_ref, masks_ref, relu=True)
    x = _conv3x3(x, w2_ref, masks_ref, relu=True)
    x = _conv3x3(x, w3_ref, masks_ref, relu=True)
    x = _conv3x3(x, w4_ref, masks_ref, relu=False)

    @pl.when(b == 0)
    def _store_tq():
        tq_scratch[...] = x

    @pl.when(b == 1)
    def _combine():
        tq = tq_scratch[...]
        tk = x
        num = jnp.sum(tk * tq, axis=0, keepdims=True)
        den = jnp.maximum(
            jnp.sqrt(jnp.sum(tk * tk, axis=0, keepdims=True))
            * jnp.sqrt(jnp.sum(tq * tq, axis=0, keepdims=True)), 1e-6)
        out_ref[...] = (num / den) * qn


def _tap_weights(w_theta, w_thetak):
    ws = jnp.stack([w_theta, w_thetak])
    ws = jnp.transpose(ws, (0, 3, 4, 1, 2))
    return ws.reshape(2, 9, w_theta.shape[0], w_theta.shape[1])


@jax.jit
def _run(query, keys, theta_w1, theta_w2, theta_w3, theta_w4,
         thetak_w1, thetak_w2, thetak_w3, thetak_w4):
    d = query.shape[1]
    qf = query.reshape(d, NT)
    qf = jnp.pad(qf, ((0, 0), (0, NP - NT)))
    masks = jnp.asarray(_MASKS)
    w1 = _tap_weights(theta_w1, thetak_w1)
    w2 = _tap_weights(theta_w2, thetak_w2)
    w3 = _tap_weights(theta_w3, thetak_w3)
    w4 = _tap_weights(theta_w4, thetak_w4)

    full = lambda shape: pl.BlockSpec(shape, lambda b: (0,) * len(shape))
    per_branch = lambda shape: pl.BlockSpec((1,) + shape[1:], lambda b: (b, 0, 0, 0))

    out = pl.pallas_call(
        _memory_body,
        grid=(2,),
        in_specs=[
            full((d, NP)),
            full((256, d)),
            full((d, 256)),
            full((16, NP)),
            per_branch(w1.shape),
            per_branch(w2.shape),
            per_branch(w3.shape),
            per_branch(w4.shape),
        ],
        out_specs=full((d, NP)),
        out_shape=jax.ShapeDtypeStruct((d, NP), jnp.float32),
        scratch_shapes=[pltpu.VMEM((64, NP), jnp.float32)],
    )(qf, keys, keys.T, masks, w1, w2, w3, w4)

    cfeature = out[:, :NT].reshape(1, d, H, W)
    return keys, cfeature


def kernel(query, keys, theta_w1, theta_w2, theta_w3, theta_w4,
           thetak_w1, thetak_w2, thetak_w3, thetak_w4, train=False):
    return _run(query, keys, theta_w1, theta_w2, theta_w3, theta_w4,
                thetak_w1, thetak_w2, thetak_w3, thetak_w4)


# speedup vs baseline: 1.5695x; 1.5695x over previous
import jax
import jax.numpy as jnp
import numpy as np
from jax.experimental import pallas as pl
from jax.experimental.pallas import tpu as pltpu

H, W = 37, 75
NT = H * W
NP = 2944


def _build_masks() -> np.ndarray:
    t = np.arange(NP)
    w = t % W
    rows = [
        (t < NT).astype(np.float32),
        ((w != W - 1) & (t < NT)).astype(np.float32),
        ((w != 0) & (t < NT)).astype(np.float32),
    ]
    rows.extend(np.zeros(NP, np.float32) for _ in range(5))
    return np.stack(rows)


_MASKS = _build_masks()


def _shift(x, delta):
    if delta == 0:
        return x
    return jnp.concatenate([x[:, delta:], x[:, :delta]], axis=1)


def _conv3x3(x_bf, w_ref, masks_ref, mvalid, relu, out_bf16):
    ml = masks_ref[pl.ds(1, 1), :].astype(jnp.bfloat16)
    mr = masks_ref[pl.ds(2, 1), :].astype(jnp.bfloat16)
    xl = x_bf * ml
    xr = x_bf * mr
    acc = None
    for tap in range(9):
        di, dj = tap // 3 - 1, tap % 3 - 1
        src = x_bf if dj == 0 else (xr if dj == 1 else xl)
        xs = _shift(src, di * W + dj)
        y = jnp.dot(w_ref[0, tap], xs, preferred_element_type=jnp.float32)
        acc = y if acc is None else acc + y
    if relu:
        acc = jnp.maximum(acc, 0.0)
    acc = acc * mvalid
    return acc.astype(jnp.bfloat16) if out_bf16 else acc


def _memory_body(qf_ref, keys_ref, keys_t_ref, masks_ref, w1_ref, w2_ref,
                 w3_ref, w4_ref, out_ref, tq_scratch):
    b = pl.program_id(0)
    qf = qf_ref[...]
    norm = jnp.sqrt(jnp.sum(qf * qf, axis=0, keepdims=True))
    qn = qf / jnp.maximum(norm, 1e-12)

    keys = keys_ref[...]
    k_norm = jnp.sqrt(jnp.sum(keys * keys, axis=1, keepdims=True))
    q_norm = jnp.sqrt(jnp.sum(qn * qn, axis=0, keepdims=True))
    dots = jnp.dot(keys, qn, preferred_element_type=jnp.float32)
    cos = dots / jnp.maximum(k_norm * q_norm, 1e-6)
    e = jnp.exp(cos - jnp.max(cos, axis=0, keepdims=True))
    score = e / jnp.sum(e, axis=0, keepdims=True)
    upd = jnp.dot(keys_t_ref[...], score, preferred_element_type=jnp.float32)

    mvalid = masks_ref[pl.ds(0, 1), :]
    x = jnp.where(b == 1, upd, qn) * mvalid
    x = x.astype(jnp.bfloat16)
    x = _conv3x3(x, w1_ref, masks_ref, mvalid, relu=True, out_bf16=True)
    x = _conv3x3(x, w2_ref, masks_ref, mvalid, relu=True, out_bf16=True)
    x = _conv3x3(x, w3_ref, masks_ref, mvalid, relu=True, out_bf16=True)
    x = _conv3x3(x, w4_ref, masks_ref, mvalid, relu=False, out_bf16=False)

    @pl.when(b == 0)
    def _store_tq():
        tq_scratch[...] = x

    @pl.when(b == 1)
    def _combine():
        tq = tq_scratch[...]
        tk = x
        num = jnp.sum(tk * tq, axis=0, keepdims=True)
        den = jnp.maximum(
            jnp.sqrt(jnp.sum(tk * tk, axis=0, keepdims=True))
            * jnp.sqrt(jnp.sum(tq * tq, axis=0, keepdims=True)), 1e-6)
        out_ref[...] = (num / den) * qn


def _tap_weights(w_theta, w_thetak):
    ws = jnp.stack([w_theta, w_thetak])
    ws = jnp.transpose(ws, (0, 3, 4, 1, 2))
    return ws.reshape(2, 9, w_theta.shape[0], w_theta.shape[1]).astype(jnp.bfloat16)


@jax.jit
def _run(query, keys, theta_w1, theta_w2, theta_w3, theta_w4,
         thetak_w1, thetak_w2, thetak_w3, thetak_w4):
    d = query.shape[1]
    qf = query.reshape(d, NT)
    qf = jnp.pad(qf, ((0, 0), (0, NP - NT)))
    masks = jnp.asarray(_MASKS)
    w1 = _tap_weights(theta_w1, thetak_w1)
    w2 = _tap_weights(theta_w2, thetak_w2)
    w3 = _tap_weights(theta_w3, thetak_w3)
    w4 = _tap_weights(theta_w4, thetak_w4)

    full = lambda shape: pl.BlockSpec(shape, lambda b: (0,) * len(shape))
    per_branch = lambda shape: pl.BlockSpec((1,) + shape[1:], lambda b: (b, 0, 0, 0))

    out = pl.pallas_call(
        _memory_body,
        grid=(2,),
        in_specs=[
            full((d, NP)),
            full((256, d)),
            full((d, 256)),
            full((8, NP)),
            per_branch(w1.shape),
            per_branch(w2.shape),
            per_branch(w3.shape),
            per_branch(w4.shape),
        ],
        out_specs=full((d, NP)),
        out_shape=jax.ShapeDtypeStruct((d, NP), jnp.float32),
        scratch_shapes=[pltpu.VMEM((64, NP), jnp.float32)],
    )(qf, keys, keys.T, masks, w1, w2, w3, w4)

    cfeature = out[:, :NT].reshape(1, d, H, W)
    return keys, cfeature


def kernel(query, keys, theta_w1, theta_w2, theta_w3, theta_w4,
           thetak_w1, thetak_w2, thetak_w3, thetak_w4, train=False):
    return _run(query, keys, theta_w1, theta_w2, theta_w3, theta_w4,
                thetak_w1, thetak_w2, thetak_w3, thetak_w4)
